# Initial kernel scaffold; baseline (speedup 1.0000x reference)
import jax
import jax.numpy as jnp
from jax.experimental import pallas as pl


def kernel(inputs, table):
    raise NotImplementedError("write your pallas kernel here")


# speedup vs baseline: 1.9093x; 1.9093x over previous
import functools

import jax
import jax.numpy as jnp
from jax import lax
from jax.experimental import pallas as pl
from jax.experimental.pallas import tpu as pltpu
from jax.experimental.pallas import tpu_sc as plsc

BATCH = 4096
HIST = 200
DIM = 32
LANES = 16

NUM_CORES = 2
NUM_SUBCORES = 16
NW = NUM_CORES * NUM_SUBCORES
ROWS_PER_W = BATCH // NW

CHUNK0 = 128
CHUNK1 = HIST - CHUNK0


def _pooling_body(inputs_hbm, table_hbm, out_hbm, idx_v, rows_v, out_v, t0_v,
                  sem):
    cid = lax.axis_index("c")
    sid = lax.axis_index("s")
    wid = sid * NUM_CORES + cid
    base = wid * ROWS_PER_W

    pltpu.sync_copy(inputs_hbm.at[pl.ds(base, ROWS_PER_W), :], idx_v)
    pltpu.sync_copy(table_hbm.at[pl.ds(0, 1), :], t0_v)
    t0a = t0_v[0, 0:LANES]
    t0b = t0_v[0, LANES:DIM]

    lane = lax.iota(jnp.int32, LANES)

    def row_body(b, carry):
        c1 = pltpu.make_async_copy(
            table_hbm.at[idx_v.at[b, pl.ds(0, CHUNK0)]],
            rows_v.at[pl.ds(0, CHUNK0), :], sem)
        c2 = pltpu.make_async_copy(
            table_hbm.at[idx_v.at[b, pl.ds(CHUNK0, CHUNK1)]],
            rows_v.at[pl.ds(CHUNK0, CHUNK1), :], sem)
        c1.start()
        c2.start()

        def cnt_body(k, cz):
            chunk = idx_v[b, pl.ds(pl.multiple_of(k * LANES, LANES), LANES)]
            return cz + plsc.all_reduce_population_count(chunk == 0)

        cz = lax.fori_loop(0, (HIST // LANES), cnt_body,
                           jnp.zeros((LANES,), jnp.int32))
        tail = idx_v[b, pl.ds(HIST - LANES, LANES)]
        cz = cz + plsc.all_reduce_population_count(
            (tail == 0) & (lane >= LANES - (HIST % LANES)))
        n0 = cz.astype(jnp.float32)

        c1.wait()
        c2.wait()

        def acc_body(j, ac):
            a0, a1 = ac
            return (a0 + rows_v[j, 0:LANES], a1 + rows_v[j, LANES:DIM])

        a0, a1 = lax.fori_loop(
            0, HIST, acc_body,
            (jnp.zeros((LANES,), jnp.float32),
             jnp.zeros((LANES,), jnp.float32)))

        inv = 1.0 / jnp.maximum(jnp.float32(HIST) - n0, 1.0)
        out_v[b, 0:LANES] = (a0 - n0 * t0a) * inv
        out_v[b, LANES:DIM] = (a1 - n0 * t0b) * inv
        return carry

    lax.fori_loop(0, ROWS_PER_W, row_body, 0)

    pltpu.sync_copy(out_v, out_hbm.at[pl.ds(base, ROWS_PER_W), :])


@functools.partial(
    pl.kernel,
    mesh=plsc.VectorSubcoreMesh(core_axis_name="c", subcore_axis_name="s"),
    compiler_params=pltpu.CompilerParams(needs_layout_passes=False,
                                         use_tc_tiling_on_sc=False),
    out_type=jax.ShapeDtypeStruct((BATCH, DIM), jnp.float32),
    scratch_types=[
        pltpu.VMEM((ROWS_PER_W, HIST), jnp.int32),
        pltpu.VMEM((HIST, DIM), jnp.float32),
        pltpu.VMEM((ROWS_PER_W, DIM), jnp.float32),
        pltpu.VMEM((1, DIM), jnp.float32),
        pltpu.SemaphoreType.DMA,
    ],
)
def _pooling_kernel(inputs_hbm, table_hbm, out_hbm, idx_v, rows_v, out_v,
                    t0_v, sem):
    _pooling_body(inputs_hbm, table_hbm, out_hbm, idx_v, rows_v, out_v, t0_v,
                  sem)


def kernel(inputs, table):
    return _pooling_kernel(inputs, table)


# speedup vs baseline: 2.3170x; 1.2136x over previous
import functools

import jax
import jax.numpy as jnp
from jax import lax
from jax.experimental import pallas as pl
from jax.experimental.pallas import tpu as pltpu
from jax.experimental.pallas import tpu_sc as plsc

BATCH = 4096
HIST = 200
DIM = 32
LANES = 16

NUM_CORES = 2
NUM_SUBCORES = 16
NW = NUM_CORES * NUM_SUBCORES
ROWS_PER_W = BATCH // NW

CHUNK0 = 128
CHUNK1 = HIST - CHUNK0

ACC_UNROLL = 8
N_ACC = 4


def _pooling_body(inputs_hbm, table_hbm, out_hbm, idx_v, rows_v, out_v, t0_v,
                  sem0, sem1):
    cid = lax.axis_index("c")
    sid = lax.axis_index("s")
    wid = sid * NUM_CORES + cid
    base = wid * ROWS_PER_W

    pltpu.sync_copy(inputs_hbm.at[pl.ds(base, ROWS_PER_W), :], idx_v)
    pltpu.sync_copy(table_hbm.at[pl.ds(0, 1), :], t0_v)
    t0a = t0_v[0, 0:LANES]
    t0b = t0_v[0, LANES:DIM]

    lane = lax.iota(jnp.int32, LANES)
    sems = (sem0, sem1)

    def start_gather(b, p):
        pltpu.make_async_copy(
            table_hbm.at[idx_v.at[b, pl.ds(0, CHUNK0)]],
            rows_v.at[p, pl.ds(0, CHUNK0), :], sems[p]).start()
        pltpu.make_async_copy(
            table_hbm.at[idx_v.at[b, pl.ds(CHUNK0, CHUNK1)]],
            rows_v.at[p, pl.ds(CHUNK0, CHUNK1), :], sems[p]).start()

    def wait_gather(p):
        pltpu.make_async_copy(
            table_hbm.at[idx_v.at[0, pl.ds(0, CHUNK0)]],
            rows_v.at[p, pl.ds(0, CHUNK0), :], sems[p]).wait()
        pltpu.make_async_copy(
            table_hbm.at[idx_v.at[0, pl.ds(CHUNK0, CHUNK1)]],
            rows_v.at[p, pl.ds(CHUNK0, CHUNK1), :], sems[p]).wait()

    def count_zeros(b):
        cz = jnp.zeros((LANES,), jnp.int32)
        for k in range(HIST // LANES):
            chunk = idx_v[b, pl.ds(k * LANES, LANES)]
            cz = cz + plsc.all_reduce_population_count(chunk == 0)
        tail = idx_v[b, pl.ds(HIST - LANES, LANES)]
        cz = cz + plsc.all_reduce_population_count(
            (tail == 0) & (lane >= LANES - (HIST % LANES)))
        return cz.astype(jnp.float32)

    def accumulate(p):
        zero = jnp.zeros((LANES,), jnp.float32)

        def step(s, ac):
            accs = list(ac)
            for jj in range(ACC_UNROLL):
                j = s * ACC_UNROLL + jj
                k = jj % N_ACC
                accs[2 * k] = accs[2 * k] + rows_v[p, j, 0:LANES]
                accs[2 * k + 1] = accs[2 * k + 1] + rows_v[p, j, LANES:DIM]
            return tuple(accs)

        accs = lax.fori_loop(0, HIST // ACC_UNROLL, step, (zero,) * 2 * N_ACC)
        a0 = accs[0]
        a1 = accs[1]
        for k in range(1, N_ACC):
            a0 = a0 + accs[2 * k]
            a1 = a1 + accs[2 * k + 1]
        return a0, a1

    def finish_row(b, p):
        n0 = count_zeros(b)
        a0, a1 = accumulate(p)
        inv = 1.0 / jnp.maximum(jnp.float32(HIST) - n0, 1.0)
        out_v[b, 0:LANES] = (a0 - n0 * t0a) * inv
        out_v[b, LANES:DIM] = (a1 - n0 * t0b) * inv

    start_gather(0, 0)

    def pair_body(g, carry):
        b0 = g * 2
        start_gather(b0 + 1, 1)
        wait_gather(0)
        finish_row(b0, 0)

        @pl.when(g < ROWS_PER_W // 2 - 1)
        def _():
            start_gather(b0 + 2, 0)

        wait_gather(1)
        finish_row(b0 + 1, 1)
        return carry

    lax.fori_loop(0, ROWS_PER_W // 2, pair_body, 0)

    pltpu.sync_copy(out_v, out_hbm.at[pl.ds(base, ROWS_PER_W), :])


@functools.partial(
    pl.kernel,
    mesh=plsc.VectorSubcoreMesh(core_axis_name="c", subcore_axis_name="s"),
    compiler_params=pltpu.CompilerParams(needs_layout_passes=False,
                                         use_tc_tiling_on_sc=False),
    out_type=jax.ShapeDtypeStruct((BATCH, DIM), jnp.float32),
    scratch_types=[
        pltpu.VMEM((ROWS_PER_W, HIST), jnp.int32),
        pltpu.VMEM((2, HIST, DIM), jnp.float32),
        pltpu.VMEM((ROWS_PER_W, DIM), jnp.float32),
        pltpu.VMEM((1, DIM), jnp.float32),
        pltpu.SemaphoreType.DMA,
        pltpu.SemaphoreType.DMA,
    ],
)
def _pooling_kernel(inputs_hbm, table_hbm, out_hbm, idx_v, rows_v, out_v,
                    t0_v, sem0, sem1):
    _pooling_body(inputs_hbm, table_hbm, out_hbm, idx_v, rows_v, out_v, t0_v,
                  sem0, sem1)


def kernel(inputs, table):
    return _pooling_kernel(inputs, table)
